# baseline (speedup 1.0000x reference)
import functools

import jax
import jax.numpy as jnp
from jax import lax
from jax.experimental import pallas as pl
from jax.experimental.pallas import tpu as pltpu
from jax.experimental.pallas import tpu_sc as plsc

N_TOKENS = 32768
N_EXPERTS = 64
TOPK = 8
LANES = 16

NUM_CORES = 2
NUM_SUBCORES = 16
NW = NUM_CORES * NUM_SUBCORES
ROWS_PER_W = N_TOKENS // NW

EBLK = N_EXPERTS // 8
TBLK = N_TOKENS // 128
TBLK_PER_W = TBLK // NW
IN_WORDS_PER_W = ROWS_PER_W * N_EXPERTS
OUT_WORDS_PER_W = ROWS_PER_W * TOPK
EBLK_STRIDE_HBM = TBLK * 1024
EBLK_STRIDE_V = TBLK_PER_W * 1024
HALF_SEG = EBLK_STRIDE_V // 2
GROUPS = ROWS_PER_W // LANES

SORT8 = ((0, 1), (2, 3), (4, 5), (6, 7),
         (0, 2), (1, 3), (4, 6), (5, 7),
         (1, 2), (5, 6),
         (0, 4), (1, 5), (2, 6), (3, 7),
         (2, 4), (3, 5),
         (1, 2), (3, 4), (5, 6))
BITONIC8 = ((0, 4), (1, 5), (2, 6), (3, 7),
            (0, 2), (1, 3), (4, 6), (5, 7),
            (0, 1), (2, 3), (4, 5), (6, 7))

_mesh = plsc.VectorSubcoreMesh(
    core_axis_name="c", subcore_axis_name="s",
    num_cores=NUM_CORES, num_subcores=NUM_SUBCORES)


def _cx(l, i, j):
  hi = jnp.maximum(l[i], l[j])
  lo = jnp.minimum(l[i], l[j])
  l[i] = hi
  l[j] = lo


@functools.partial(
    pl.kernel,
    out_type=[
        jax.ShapeDtypeStruct((N_TOKENS * TOPK,), jnp.int32),
        jax.ShapeDtypeStruct((N_TOKENS * TOPK,), jnp.float32),
    ],
    mesh=_mesh,
    scratch_types=[
        pltpu.VMEM((IN_WORDS_PER_W,), jnp.float32),
        pltpu.VMEM((OUT_WORDS_PER_W,), jnp.int32),
        pltpu.VMEM((OUT_WORDS_PER_W,), jnp.float32),
        pltpu.SemaphoreType.DMA,
        pltpu.SemaphoreType.DMA,
        pltpu.SemaphoreType.DMA,
    ],
    compiler_params=pltpu.CompilerParams(needs_layout_passes=False),
)
def _route(logits_hbm, out_idx_hbm, out_val_hbm, logits_v, idx_v, val_v,
           sem0, sem1, osem):
  wid = lax.axis_index("s") * NUM_CORES + lax.axis_index("c")
  tb0 = wid * TBLK_PER_W
  half_copies = []
  for h, sem in ((0, sem0), (1, sem1)):
    half_copies.append([
        pltpu.async_copy(
            logits_hbm.at[pl.ds(
                b * EBLK_STRIDE_HBM + tb0 * 1024 + h * HALF_SEG, HALF_SEG)],
            logits_v.at[pl.ds(b * EBLK_STRIDE_V + h * HALF_SEG, HALF_SEG)],
            sem)
        for b in range(EBLK)
    ])

  def body(g):
    t0 = g * LANES
    goff = (t0 >> 7) * 896 + t0
    denom = None
    keys = []
    for e in range(N_EXPERTS):
      eoff = (e >> 3) * EBLK_STRIDE_V + (e & 7) * 128
      x = logits_v[pl.ds(goff + eoff, LANES)]
      ex = jnp.exp(x)
      denom = ex if denom is None else denom + ex
      kb = plsc.bitcast(ex, jnp.int32)
      keys.append(plsc.bitcast((kb & -64) | (63 - e), jnp.float32))

    run = None
    for r in range(EBLK):
      grp = keys[8 * r:8 * r + 8]
      for i, j in SORT8:
        _cx(grp, i, j)
      if run is None:
        run = grp
      else:
        run = [jnp.maximum(run[j], grp[7 - j]) for j in range(TOPK)]
        for i, j in BITONIC8:
          _cx(run, i, j)

    inv = 1.0 / denom
    for k in range(TOPK):
      kb = plsc.bitcast(run[k], jnp.int32)
      o = pl.ds(goff + k * 128, LANES)
      idx_v[o] = 63 - (kb & 63)
      val_v[o] = plsc.bitcast(kb & -64, jnp.float32) * inv

  for c in half_copies[0]:
    c.wait()
  plsc.parallel_loop(0, GROUPS // 2, 1, unroll=4)(body)
  oc0 = pltpu.async_copy(idx_v.at[pl.ds(0, OUT_WORDS_PER_W // 2)],
                         out_idx_hbm.at[pl.ds(wid * OUT_WORDS_PER_W,
                                              OUT_WORDS_PER_W // 2)],
                         osem)
  oc1 = pltpu.async_copy(val_v.at[pl.ds(0, OUT_WORDS_PER_W // 2)],
                         out_val_hbm.at[pl.ds(wid * OUT_WORDS_PER_W,
                                              OUT_WORDS_PER_W // 2)],
                         osem)
  for c in half_copies[1]:
    c.wait()
  plsc.parallel_loop(GROUPS // 2, GROUPS, 1, unroll=4)(body)
  oc2 = pltpu.async_copy(
      idx_v.at[pl.ds(OUT_WORDS_PER_W // 2, OUT_WORDS_PER_W // 2)],
      out_idx_hbm.at[pl.ds(wid * OUT_WORDS_PER_W + OUT_WORDS_PER_W // 2,
                           OUT_WORDS_PER_W // 2)],
      osem)
  oc3 = pltpu.async_copy(
      val_v.at[pl.ds(OUT_WORDS_PER_W // 2, OUT_WORDS_PER_W // 2)],
      out_val_hbm.at[pl.ds(wid * OUT_WORDS_PER_W + OUT_WORDS_PER_W // 2,
                           OUT_WORDS_PER_W // 2)],
      osem)
  oc0.wait()
  oc1.wait()
  oc2.wait()
  oc3.wait()


def kernel(router_logits):
  x_flat = (router_logits.T
            .reshape(EBLK, 8, TBLK, 128)
            .transpose(0, 2, 1, 3)
            .reshape(-1))
  idx_flat, val_flat = _route(x_flat)
  idx = idx_flat.reshape(TBLK, TOPK, 128).transpose(0, 2, 1).reshape(
      N_TOKENS, TOPK)
  val = val_flat.reshape(TBLK, TOPK, 128).transpose(0, 2, 1).reshape(
      N_TOKENS, TOPK)
  return (idx, val)


# speedup vs baseline: 1.0273x; 1.0273x over previous
import functools

import jax
import jax.numpy as jnp
from jax import lax
from jax.experimental import pallas as pl
from jax.experimental.pallas import tpu as pltpu
from jax.experimental.pallas import tpu_sc as plsc

N_TOKENS = 32768
N_EXPERTS = 64
TOPK = 8
LANES = 16

NUM_CORES = 2
NUM_SUBCORES = 16
NW = NUM_CORES * NUM_SUBCORES
ROWS_PER_W = N_TOKENS // NW

EBLK = N_EXPERTS // 8
TBLK = N_TOKENS // 128
TBLK_PER_W = TBLK // NW
IN_WORDS_PER_W = ROWS_PER_W * N_EXPERTS
OUT_WORDS_PER_W = ROWS_PER_W * TOPK
EBLK_STRIDE_HBM = TBLK * 1024
EBLK_STRIDE_V = TBLK_PER_W * 1024
HALF_SEG = EBLK_STRIDE_V // 2
GROUPS = ROWS_PER_W // LANES

SORT8 = ((0, 1), (2, 3), (4, 5), (6, 7),
         (0, 2), (1, 3), (4, 6), (5, 7),
         (1, 2), (5, 6),
         (0, 4), (1, 5), (2, 6), (3, 7),
         (2, 4), (3, 5),
         (1, 2), (3, 4), (5, 6))
BITONIC8 = ((0, 4), (1, 5), (2, 6), (3, 7),
            (0, 2), (1, 3), (4, 6), (5, 7),
            (0, 1), (2, 3), (4, 5), (6, 7))

_mesh = plsc.VectorSubcoreMesh(
    core_axis_name="c", subcore_axis_name="s",
    num_cores=NUM_CORES, num_subcores=NUM_SUBCORES)


def _cx(l, i, j):
  hi = jnp.maximum(l[i], l[j])
  lo = jnp.minimum(l[i], l[j])
  l[i] = hi
  l[j] = lo


@functools.partial(
    pl.kernel,
    out_type=[
        jax.ShapeDtypeStruct((N_TOKENS * TOPK,), jnp.int32),
        jax.ShapeDtypeStruct((N_TOKENS * TOPK,), jnp.float32),
    ],
    mesh=_mesh,
    scratch_types=[
        pltpu.VMEM((IN_WORDS_PER_W,), jnp.float32),
        pltpu.VMEM((OUT_WORDS_PER_W,), jnp.int32),
        pltpu.VMEM((OUT_WORDS_PER_W,), jnp.float32),
        pltpu.SemaphoreType.DMA,
        pltpu.SemaphoreType.DMA,
        pltpu.SemaphoreType.DMA,
    ],
    compiler_params=pltpu.CompilerParams(needs_layout_passes=False),
)
def _route(logits_hbm, out_idx_hbm, out_val_hbm, logits_v, idx_v, val_v,
           sem0, sem1, osem):
  wid = lax.axis_index("s") * NUM_CORES + lax.axis_index("c")
  tb0 = wid * TBLK_PER_W
  QSEG = EBLK_STRIDE_V // 4
  quarter_copies = []
  for q in range(4):
    quarter_copies.append([
        pltpu.async_copy(
            logits_hbm.at[pl.ds(
                b * EBLK_STRIDE_HBM + tb0 * 1024 + q * QSEG, QSEG)],
            logits_v.at[pl.ds(b * EBLK_STRIDE_V + q * QSEG, QSEG)],
            sem0 if q % 2 == 0 else sem1)
        for b in range(EBLK)
    ])

  def body(g):
    t0 = g * LANES
    goff = (t0 >> 7) * 896 + t0
    denom = None
    keys = []
    for e in range(N_EXPERTS):
      eoff = (e >> 3) * EBLK_STRIDE_V + (e & 7) * 128
      x = logits_v[pl.ds(goff + eoff, LANES)]
      ex = jnp.exp(x)
      denom = ex if denom is None else denom + ex
      kb = plsc.bitcast(ex, jnp.int32)
      keys.append(plsc.bitcast((kb & -64) | (63 - e), jnp.float32))

    run = None
    for r in range(EBLK):
      grp = keys[8 * r:8 * r + 8]
      for i, j in SORT8:
        _cx(grp, i, j)
      if run is None:
        run = grp
      else:
        run = [jnp.maximum(run[j], grp[7 - j]) for j in range(TOPK)]
        for i, j in BITONIC8:
          _cx(run, i, j)

    inv = 1.0 / denom
    for k in range(TOPK):
      kb = plsc.bitcast(run[k], jnp.int32)
      o = pl.ds(goff + k * 128, LANES)
      idx_v[o] = 63 - (kb & 63)
      val_v[o] = plsc.bitcast(kb & -64, jnp.float32) * inv

  OUTQ = OUT_WORDS_PER_W // 4
  out_copies = []
  for q in range(4):
    for c in quarter_copies[q]:
      c.wait()
    plsc.parallel_loop(q * (GROUPS // 4), (q + 1) * (GROUPS // 4), 1,
                       unroll=2)(body)
    out_copies.append(pltpu.async_copy(
        idx_v.at[pl.ds(q * OUTQ, OUTQ)],
        out_idx_hbm.at[pl.ds(wid * OUT_WORDS_PER_W + q * OUTQ, OUTQ)],
        osem))
    out_copies.append(pltpu.async_copy(
        val_v.at[pl.ds(q * OUTQ, OUTQ)],
        out_val_hbm.at[pl.ds(wid * OUT_WORDS_PER_W + q * OUTQ, OUTQ)],
        osem))
  for c in out_copies:
    c.wait()


def kernel(router_logits):
  x_flat = (router_logits.T
            .reshape(EBLK, 8, TBLK, 128)
            .transpose(0, 2, 1, 3)
            .reshape(-1))
  idx_flat, val_flat = _route(x_flat)
  idx = idx_flat.reshape(TBLK, TOPK, 128).transpose(0, 2, 1).reshape(
      N_TOKENS, TOPK)
  val = val_flat.reshape(TBLK, TOPK, 128).transpose(0, 2, 1).reshape(
      N_TOKENS, TOPK)
  return (idx, val)


# speedup vs baseline: 1.1213x; 1.0915x over previous
import functools

import jax
import jax.numpy as jnp
from jax import lax
from jax.experimental import pallas as pl
from jax.experimental.pallas import tpu as pltpu
from jax.experimental.pallas import tpu_sc as plsc

N_TOKENS = 32768
N_EXPERTS = 64
TOPK = 8
LANES = 16

NUM_CORES = 2
NUM_SUBCORES = 16
NW = NUM_CORES * NUM_SUBCORES
ROWS_PER_W = N_TOKENS // NW

EBLK = N_EXPERTS // 8
TBLK = N_TOKENS // 128
TBLK_PER_W = TBLK // NW
IN_WORDS_PER_W = ROWS_PER_W * N_EXPERTS
OUT_WORDS_PER_W = ROWS_PER_W * TOPK
EBLK_STRIDE_HBM = TBLK * 1024
EBLK_STRIDE_V = TBLK_PER_W * 1024
HALF_SEG = EBLK_STRIDE_V // 2
GROUPS = ROWS_PER_W // LANES

SORT8 = ((0, 1), (2, 3), (4, 5), (6, 7),
         (0, 2), (1, 3), (4, 6), (5, 7),
         (1, 2), (5, 6),
         (0, 4), (1, 5), (2, 6), (3, 7),
         (2, 4), (3, 5),
         (1, 2), (3, 4), (5, 6))
BITONIC8 = ((0, 4), (1, 5), (2, 6), (3, 7),
            (0, 2), (1, 3), (4, 6), (5, 7),
            (0, 1), (2, 3), (4, 5), (6, 7))

_mesh = plsc.VectorSubcoreMesh(
    core_axis_name="c", subcore_axis_name="s",
    num_cores=NUM_CORES, num_subcores=NUM_SUBCORES)


def _cx(l, i, j):
  hi = jnp.maximum(l[i], l[j])
  lo = jnp.minimum(l[i], l[j])
  l[i] = hi
  l[j] = lo


@functools.partial(
    pl.kernel,
    out_type=[
        jax.ShapeDtypeStruct((N_TOKENS * TOPK,), jnp.int32),
        jax.ShapeDtypeStruct((N_TOKENS * TOPK,), jnp.float32),
    ],
    mesh=_mesh,
    scratch_types=[
        pltpu.VMEM((IN_WORDS_PER_W,), jnp.float32),
        pltpu.VMEM((OUT_WORDS_PER_W,), jnp.int32),
        pltpu.VMEM((OUT_WORDS_PER_W,), jnp.float32),
        pltpu.SemaphoreType.DMA,
        pltpu.SemaphoreType.DMA,
        pltpu.SemaphoreType.DMA,
    ],
    compiler_params=pltpu.CompilerParams(needs_layout_passes=False),
)
def _route(logits_hbm, out_idx_hbm, out_val_hbm, logits_v, idx_v, val_v,
           sem0, sem1, osem):
  wid = lax.axis_index("s") * NUM_CORES + lax.axis_index("c")
  tb0 = wid * TBLK_PER_W
  QSEG = EBLK_STRIDE_V // 4
  quarter_copies = []
  for q in range(4):
    quarter_copies.append([
        pltpu.async_copy(
            logits_hbm.at[pl.ds(
                b * EBLK_STRIDE_HBM + tb0 * 1024 + q * QSEG, QSEG)],
            logits_v.at[pl.ds(b * EBLK_STRIDE_V + q * QSEG, QSEG)],
            sem0 if q % 2 == 0 else sem1)
        for b in range(EBLK)
    ])

  def body(g):
    t0 = g * LANES
    goff = (t0 >> 7) * 896 + t0
    denom = None
    keys = []
    for e in range(N_EXPERTS):
      eoff = (e >> 3) * EBLK_STRIDE_V + (e & 7) * 128
      x = logits_v[pl.ds(goff + eoff, LANES)]
      ex = jnp.exp(x)
      denom = ex if denom is None else denom + ex
      kb = plsc.bitcast(ex, jnp.int32)
      keys.append(plsc.bitcast((kb & -64) | (63 - e), jnp.float32))

    run = None
    for r in range(EBLK):
      grp = keys[8 * r:8 * r + 8]
      for i, j in SORT8:
        _cx(grp, i, j)
      if run is None:
        run = grp
      else:
        run = [jnp.maximum(run[j], grp[7 - j]) for j in range(TOPK)]
        for i, j in BITONIC8:
          _cx(run, i, j)

    inv = 1.0 / denom
    for k in range(TOPK):
      kb = plsc.bitcast(run[k], jnp.int32)
      o = pl.ds(goff + k * 128, LANES)
      idx_v[o] = 63 - (kb & 63)
      val_v[o] = plsc.bitcast(kb & -64, jnp.float32) * inv

  for q in range(4):
    for c in quarter_copies[q]:
      c.wait()
  plsc.parallel_loop(0, GROUPS, 1, unroll=2)(body)
  oc0 = pltpu.async_copy(idx_v,
                         out_idx_hbm.at[pl.ds(wid * OUT_WORDS_PER_W,
                                              OUT_WORDS_PER_W)],
                         osem)
  oc1 = pltpu.async_copy(val_v,
                         out_val_hbm.at[pl.ds(wid * OUT_WORDS_PER_W,
                                              OUT_WORDS_PER_W)],
                         osem)
  oc0.wait()
  oc1.wait()


def kernel(router_logits):
  x_flat = (router_logits.T
            .reshape(EBLK, 8, TBLK, 128)
            .transpose(0, 2, 1, 3)
            .reshape(-1))
  idx_flat, val_flat = _route(x_flat)
  idx = idx_flat.reshape(TBLK, TOPK, 128).transpose(0, 2, 1).reshape(
      N_TOKENS, TOPK)
  val = val_flat.reshape(TBLK, TOPK, 128).transpose(0, 2, 1).reshape(
      N_TOKENS, TOPK)
  return (idx, val)
